# baseline (speedup 1.0000x reference)
import functools

import jax
import jax.numpy as jnp
from jax import lax
from jax.experimental import pallas as pl
from jax.experimental.pallas import tpu as pltpu
from jax.experimental.pallas import tpu_sc as plsc

NC = 2
NS = 16
NW = NC * NS

B = 8
T = 4
BT = B * T
N = 1601
D = 1280
CK = 16
NFULL = N // CK
VPT = D // 16


def _sc_add_pe(x3, gflat, lpe, p_arr, m_arr, c1_arr):
  mesh = plsc.VectorSubcoreMesh(core_axis_name="c", subcore_axis_name="s")

  @functools.partial(
      pl.kernel,
      mesh=mesh,
      out_type=jax.ShapeDtypeStruct((BT, N, D), jnp.float32),
      scratch_types=[
          pltpu.VMEM((CK, D), jnp.float32),
          pltpu.VMEM((CK, D), jnp.float32),
          pltpu.VMEM((CK, D), jnp.float32),
          pltpu.VMEM((BT, 16), jnp.int32),
          pltpu.VMEM((BT, 16), jnp.float32),
          pltpu.VMEM((16,), jnp.float32),
          pltpu.SemaphoreType.DMA,
      ],
  )
  def k(x_hbm, g_hbm, l_hbm, p_hbm, m_hbm, c1_hbm, out_hbm,
        xb, gb, lb, pb, mb, c1b, sem):
    wid = lax.axis_index("s") * NC + lax.axis_index("c")
    pltpu.sync_copy(p_hbm, pb)
    pltpu.sync_copy(m_hbm, mb)
    pltpu.sync_copy(c1_hbm, c1b)
    c1v = c1b[:]
    iota = lax.iota(jnp.int32, 16)
    lo = (wid * NFULL) // NW
    hi = ((wid + 1) * NFULL) // NW

    def chunk_body(c, carry):
      n0 = c * CK
      pltpu.sync_copy(l_hbm.at[pl.ds(n0, CK), :], lb)

      def bt_body(bt, carry2):
        idx = pb[bt, :] * N + (n0 + iota)
        cp = pltpu.async_copy(g_hbm.at[idx], gb, sem)
        pltpu.sync_copy(x_hbm.at[bt, pl.ds(n0, CK), :], xb)
        cp.wait()
        mv = mb[bt, :]

        def i_body(i, c3):
          def j_body(j, c4):
            s = pl.ds(j * 16, 16)
            xb[i, s] = xb[i, s] + c1v * lb[i, s] + mv * gb[i, s]
            return c4
          return lax.fori_loop(0, VPT, j_body, c3)

        lax.fori_loop(0, CK, i_body, 0)
        pltpu.sync_copy(xb, out_hbm.at[bt, pl.ds(n0, CK), :])
        return carry2

      lax.fori_loop(0, BT, bt_body, 0)
      return carry

    lax.fori_loop(lo, hi, chunk_body, 0)

    @pl.when(wid == 0)
    def _():
      n0 = NFULL * CK
      pltpu.sync_copy(l_hbm.at[pl.ds(n0, 1), :], lb.at[pl.ds(0, 1), :])

      def bt_tail(bt, carry2):
        idx = pb[bt, :] * N + n0
        cp = pltpu.async_copy(g_hbm.at[idx], gb, sem)
        pltpu.sync_copy(x_hbm.at[bt, pl.ds(n0, 1), :], xb.at[pl.ds(0, 1), :])
        cp.wait()
        mv = mb[bt, :]

        def j_body(j, c4):
          s = pl.ds(j * 16, 16)
          xb[0, s] = xb[0, s] + c1v * lb[0, s] + mv * gb[0, s]
          return c4

        lax.fori_loop(0, VPT, j_body, 0)
        pltpu.sync_copy(xb.at[pl.ds(0, 1), :], out_hbm.at[bt, pl.ds(n0, 1), :])
        return carry2

      lax.fori_loop(0, BT, bt_tail, 0)

  return k(x3, gflat, lpe, p_arr, m_arr, c1_arr)


def kernel(x, aspect_ratio, local_pe, global_pe, gate):
  b, t, n, d = x.shape
  g2 = jnp.tanh(gate[0].astype(jnp.float32))
  c1 = 1.0 - g2
  ar = aspect_ratio.astype(jnp.int32)
  h = ar[:, 0]
  w = ar[:, 1]
  wsafe = jnp.maximum(w, 1)
  tt = jnp.arange(T, dtype=jnp.int32)
  rows = tt[None, :] // wsafe[:, None]
  cols = tt[None, :] % wsafe[:, None]
  plane = (rows * T + cols).reshape(BT)
  valid = (tt[None, :] < (h * w)[:, None]).reshape(BT)
  p_arr = jnp.tile(plane.reshape(BT, 1), (1, 16))
  m_arr = jnp.tile((g2 * valid.astype(jnp.float32)).reshape(BT, 1), (1, 16))
  c1_arr = jnp.full((16,), c1, jnp.float32)
  x3 = x.reshape(BT, N, D)
  gflat = global_pe.reshape(T * T * N, D)
  out = _sc_add_pe(x3, gflat, local_pe, p_arr, m_arr, c1_arr)
  return out.reshape(b, t, n, d)
